# baseline (speedup 1.0000x reference)
import functools

import jax
import jax.numpy as jnp
from jax import lax
from jax.experimental import pallas as pl
from jax.experimental.pallas import tpu as pltpu
from jax.experimental.pallas import tpu_sc as plsc

NC, NS, L = 2, 16, 16
NW = NC * NS


@functools.lru_cache(maxsize=None)
def _build(B, N, C, K):
    nodes = B * N
    per_w = nodes // NW
    chunk = 5
    nchunk = per_w // chunk
    outrow = 2 * C * K
    assert per_w * NW == nodes and nchunk * chunk == per_w
    assert chunk * K <= 128 and C % L == 0

    mesh = plsc.VectorSubcoreMesh(
        core_axis_name="c", subcore_axis_name="s",
        num_cores=NC, num_subcores=NS)

    @functools.partial(
        pl.kernel,
        out_type=jax.ShapeDtypeStruct((nodes * outrow,), jnp.float32),
        mesh=mesh,
        compiler_params=pltpu.CompilerParams(needs_layout_passes=False),
        scratch_types=[
            pltpu.VMEM((chunk * K + L,), jnp.int32),
            pltpu.VMEM((chunk * K + L, C), jnp.float32),
            pltpu.VMEM((chunk * outrow,), jnp.float32),
            pltpu.SemaphoreType.DMA,
        ],
    )
    def ldg(points_hbm, idx_hbm, out_hbm, idx_v, g_v, o_v, sem):
        wid = lax.axis_index("s") * NC + lax.axis_index("c")
        base0 = wid * per_w
        boff = jnp.where(base0 >= N, jnp.int32(N), jnp.int32(0))
        iot = lax.iota(jnp.int32, L)

        def chunk_body(ci, _):
            nb = base0 + ci * chunk
            pltpu.sync_copy(
                idx_hbm.at[pl.ds(nb * K, chunk * K)],
                idx_v.at[pl.ds(0, chunk * K)])
            for r in range(chunk):
                sl = pl.ds(r * L, L)
                idx_v[sl] = idx_v[sl] + boff
            idx_v[pl.ds(chunk * K, L)] = jnp.minimum(
                nb + iot, jnp.int32(nodes - 1))
            pltpu.async_copy(points_hbm.at[idx_v], g_v, sem).wait()

            def ch_body(c, _):
                cols = jnp.full((L,), c, jnp.int32)
                for i in range(chunk):
                    gv = plsc.load_gather(g_v, [i * K + iot, cols])
                    xv = plsc.load_gather(
                        g_v, [jnp.full((L,), chunk * K + i, jnp.int32), cols])
                    o_v[pl.ds(i * outrow + c * K, K)] = gv - xv
                    o_v[pl.ds(i * outrow + C * K + c * K, K)] = xv
                return 0

            lax.fori_loop(0, C, ch_body, 0)
            pltpu.sync_copy(o_v, out_hbm.at[pl.ds(nb * outrow, chunk * outrow)])
            return 0

        lax.fori_loop(0, nchunk, chunk_body, 0)

    return ldg


def kernel(points, idx):
    B, N, C = points.shape
    K = idx.shape[2]
    ldg = _build(B, N, C, K)
    out = ldg(points.reshape(B * N, C), idx.reshape(-1))
    return out.reshape(B, N, 2 * C, K)


# speedup vs baseline: 3.1595x; 3.1595x over previous
import functools

import jax
import jax.numpy as jnp
from jax import lax
from jax.experimental import pallas as pl
from jax.experimental.pallas import tpu as pltpu
from jax.experimental.pallas import tpu_sc as plsc

NC, NS, L = 2, 16, 16
NW = NC * NS


@functools.lru_cache(maxsize=None)
def _build(B, N, C, K):
    nodes = B * N
    per_w = nodes // NW
    chunk = 5
    nchunk = per_w // chunk
    c2 = 2 * C
    outrow = K * c2
    assert per_w * NW == nodes and nchunk * chunk == per_w
    assert chunk * K + L <= 128 and C % L == 0

    mesh = plsc.VectorSubcoreMesh(
        core_axis_name="c", subcore_axis_name="s",
        num_cores=NC, num_subcores=NS)

    @functools.partial(
        pl.kernel,
        out_type=jax.ShapeDtypeStruct((nodes * outrow,), jnp.float32),
        mesh=mesh,
        compiler_params=pltpu.CompilerParams(needs_layout_passes=False),
        scratch_types=[
            pltpu.VMEM((chunk * K + L,), jnp.int32),
            pltpu.VMEM((chunk * K + L, C), jnp.float32),
            pltpu.VMEM((chunk * outrow,), jnp.float32),
            pltpu.SemaphoreType.DMA,
        ],
    )
    def ldg(points_hbm, idx_hbm, out_hbm, idx_v, g_v, o_v, sem):
        wid = lax.axis_index("s") * NC + lax.axis_index("c")
        base0 = wid * per_w
        boff = jnp.where(base0 >= N, jnp.int32(N), jnp.int32(0))
        iot = lax.iota(jnp.int32, L)

        def chunk_body(ci, _):
            nb = base0 + ci * chunk
            pltpu.sync_copy(
                idx_hbm.at[pl.ds(nb * K, chunk * K)],
                idx_v.at[pl.ds(0, chunk * K)])
            for r in range(chunk):
                sl = pl.ds(r * L, L)
                idx_v[sl] = idx_v[sl] + boff
            idx_v[pl.ds(chunk * K, L)] = jnp.minimum(
                nb + iot, jnp.int32(nodes - 1))
            pltpu.async_copy(points_hbm.at[idx_v], g_v, sem).wait()

            for i in range(chunk):
                xs = [g_v[chunk * K + i, pl.ds(cc * L, L)]
                      for cc in range(C // L)]
                obase = i * outrow

                def j_body(j, _, i=i, xs=xs, obase=obase):
                    row = i * K + j
                    dbase = obase + j * c2
                    for cc in range(C // L):
                        gv = g_v[row, pl.ds(cc * L, L)]
                        o_v[pl.ds(dbase + cc * L, L)] = gv - xs[cc]
                        o_v[pl.ds(dbase + C + cc * L, L)] = xs[cc]
                    return 0

                lax.fori_loop(0, K, j_body, 0)

            pltpu.sync_copy(o_v, out_hbm.at[pl.ds(nb * outrow, chunk * outrow)])
            return 0

        lax.fori_loop(0, nchunk, chunk_body, 0)

    return ldg


def kernel(points, idx):
    B, N, C = points.shape
    K = idx.shape[2]
    ldg = _build(B, N, C, K)
    out = ldg(points.reshape(B * N, C), idx.reshape(-1))
    return out.reshape(B, N, K, 2 * C).transpose(0, 1, 3, 2)


# speedup vs baseline: 4.4139x; 1.3970x over previous
import functools

import jax
import jax.numpy as jnp
from jax import lax
from jax.experimental import pallas as pl
from jax.experimental.pallas import tpu as pltpu
from jax.experimental.pallas import tpu_sc as plsc

NC, NS, L = 2, 16, 16
NW = NC * NS


@functools.lru_cache(maxsize=None)
def _build(B, N, C, K):
    nodes = B * N
    per_w = nodes // NW
    chunk = 5
    nchunk = per_w // chunk
    c2 = 2 * C
    outrow = K * c2
    assert per_w * NW == nodes and nchunk * chunk == per_w
    assert chunk * K + L <= 128 and C % L == 0

    mesh = plsc.VectorSubcoreMesh(
        core_axis_name="c", subcore_axis_name="s",
        num_cores=NC, num_subcores=NS)

    @functools.partial(
        pl.kernel,
        out_type=jax.ShapeDtypeStruct((nodes, K, c2), jnp.float32),
        mesh=mesh,
        compiler_params=pltpu.CompilerParams(needs_layout_passes=False),
        scratch_types=[
            pltpu.VMEM((chunk * K + L,), jnp.int32),
            pltpu.VMEM((chunk * K + L, C), jnp.float32),
            pltpu.VMEM((chunk, K, c2), jnp.float32),
            pltpu.SemaphoreType.DMA,
        ],
    )
    def ldg(points_hbm, idx_hbm, out_hbm, idx_v, g_v, o_v, sem):
        wid = lax.axis_index("s") * NC + lax.axis_index("c")
        base0 = wid * per_w
        boff = jnp.where(base0 >= N, jnp.int32(N), jnp.int32(0))
        iot = lax.iota(jnp.int32, L)

        def chunk_body(ci, _):
            nb = base0 + ci * chunk
            pltpu.sync_copy(
                idx_hbm.at[pl.ds(nb * K, chunk * K)],
                idx_v.at[pl.ds(0, chunk * K)])
            for r in range(chunk):
                sl = pl.ds(r * L, L)
                idx_v[sl] = idx_v[sl] + boff
            idx_v[pl.ds(chunk * K, L)] = jnp.minimum(
                nb + iot, jnp.int32(nodes - 1))
            pltpu.async_copy(points_hbm.at[idx_v], g_v, sem).wait()

            for i in range(chunk):
                xs = [g_v[chunk * K + i, pl.ds(cc * L, L)]
                      for cc in range(C // L)]

                def j_body(j, _, i=i, xs=xs):
                    row = i * K + j
                    for cc in range(C // L):
                        gv = g_v[row, pl.ds(cc * L, L)]
                        o_v[i, j, pl.ds(cc * L, L)] = gv - xs[cc]
                        o_v[i, j, pl.ds(C + cc * L, L)] = xs[cc]
                    return 0

                lax.fori_loop(0, K, j_body, 0)

            pltpu.sync_copy(o_v, out_hbm.at[pl.ds(nb, chunk)])
            return 0

        lax.fori_loop(0, nchunk, chunk_body, 0)

    return ldg


def kernel(points, idx):
    B, N, C = points.shape
    K = idx.shape[2]
    ldg = _build(B, N, C, K)
    out = ldg(points.reshape(B * N, C), idx.reshape(-1))
    return out.reshape(B, N, K, 2 * C).transpose(0, 1, 3, 2)


# speedup vs baseline: 7.4280x; 1.6829x over previous
import functools

import jax
import jax.numpy as jnp
from jax import lax
from jax.experimental import pallas as pl
from jax.experimental.pallas import tpu as pltpu
from jax.experimental.pallas import tpu_sc as plsc

NC, NS, L = 2, 16, 16
NW = NC * NS


@functools.lru_cache(maxsize=None)
def _build(B, N, C, K):
    nodes = B * N
    per_w = nodes // NW
    chunk = 5
    nchunk = per_w // chunk
    c2 = 2 * C
    glen = chunk * K + L
    assert per_w * NW == nodes and nchunk * chunk == per_w
    assert glen <= 128 and C % L == 0

    mesh = plsc.VectorSubcoreMesh(
        core_axis_name="c", subcore_axis_name="s",
        num_cores=NC, num_subcores=NS)

    @functools.partial(
        pl.kernel,
        out_type=jax.ShapeDtypeStruct((nodes, K, c2), jnp.float32),
        mesh=mesh,
        compiler_params=pltpu.CompilerParams(needs_layout_passes=False),
        scratch_types=[
            pltpu.VMEM((per_w * K,), jnp.int32),
            pltpu.VMEM((nchunk, glen), jnp.int32),
            pltpu.VMEM((glen, C), jnp.float32),
            pltpu.VMEM((glen, C), jnp.float32),
            pltpu.VMEM((chunk, K, c2), jnp.float32),
            pltpu.VMEM((chunk, K, c2), jnp.float32),
            pltpu.SemaphoreType.DMA,
            pltpu.SemaphoreType.DMA,
            pltpu.SemaphoreType.DMA,
            pltpu.SemaphoreType.DMA,
        ],
    )
    def ldg(points_hbm, idx_hbm, out_hbm,
            midx_v, cidx_v, g0_v, g1_v, o0_v, o1_v, gs0, gs1, ws0, ws1):
        g_v, o_v, gs, ws = (g0_v, g1_v), (o0_v, o1_v), (gs0, gs1), (ws0, ws1)
        wid = lax.axis_index("s") * NC + lax.axis_index("c")
        base0 = wid * per_w
        boff = jnp.where(base0 >= N, jnp.int32(N), jnp.int32(0))
        iot = lax.iota(jnp.int32, L)

        pltpu.sync_copy(idx_hbm.at[pl.ds(base0 * K, per_w * K)], midx_v)

        def build_body(ci, _):
            for r in range(chunk):
                cidx_v[ci, pl.ds(r * L, L)] = (
                    midx_v[pl.ds(ci * (chunk * K) + r * L, L)] + boff)
            cidx_v[ci, pl.ds(chunk * K, L)] = jnp.minimum(
                base0 + ci * chunk + iot, jnp.int32(nodes - 1))
            return 0

        lax.fori_loop(0, nchunk, build_body, 0)

        def fire_gather(ci, b):
            pltpu.async_copy(points_hbm.at[cidx_v.at[ci]], g_v[b], gs[b])

        def wait_gather(ci, b):
            pltpu.make_async_copy(
                points_hbm.at[cidx_v.at[ci]], g_v[b], gs[b]).wait()

        def out_slice(ci):
            return out_hbm.at[pl.ds(base0 + ci * chunk, chunk)]

        fire_gather(0, 0)

        def pair_body(it, _):
            for b in range(2):
                ci = it * 2 + b

                @pl.when(ci < nchunk)
                def _(ci=ci, b=b):
                    @pl.when(ci + 1 < nchunk)
                    def _():
                        fire_gather(ci + 1, 1 - b)

                    wait_gather(ci, b)

                    @pl.when(ci >= 2)
                    def _():
                        pltpu.make_async_copy(
                            o_v[b], out_slice(ci - 2), ws[b]).wait()

                    gb, ob = g_v[b], o_v[b]
                    for i in range(chunk):
                        xs = [gb[chunk * K + i, pl.ds(cc * L, L)]
                              for cc in range(C // L)]

                        def j_body(j, _, i=i, xs=xs):
                            row = i * K + j
                            for cc in range(C // L):
                                gv = gb[row, pl.ds(cc * L, L)]
                                ob[i, j, pl.ds(cc * L, L)] = gv - xs[cc]
                                ob[i, j, pl.ds(C + cc * L, L)] = xs[cc]
                            return 0

                        lax.fori_loop(0, K, j_body, 0)

                    pltpu.async_copy(o_v[b], out_slice(ci), ws[b])
            return 0

        lax.fori_loop(0, (nchunk + 1) // 2, pair_body, 0)
        pltpu.make_async_copy(o_v[1], out_slice(nchunk - 2), ws[1]).wait()
        pltpu.make_async_copy(o_v[0], out_slice(nchunk - 1), ws[0]).wait()

    return ldg


def kernel(points, idx):
    B, N, C = points.shape
    K = idx.shape[2]
    ldg = _build(B, N, C, K)
    out = ldg(points.reshape(B * N, C), idx.reshape(-1))
    return out.reshape(B, N, K, 2 * C).transpose(0, 1, 3, 2)


# speedup vs baseline: 13.0301x; 1.7542x over previous
import functools

import jax
import jax.numpy as jnp
from jax import lax
from jax.experimental import pallas as pl
from jax.experimental.pallas import tpu as pltpu
from jax.experimental.pallas import tpu_sc as plsc

NC, NS, L = 2, 16, 16
NW = NC * NS


@functools.lru_cache(maxsize=None)
def _build(B, N, C, K):
    nodes = B * N
    per_w = nodes // NW
    chunk = 5
    nchunk = per_w // chunk
    c2 = 2 * C
    glen = chunk * K + L
    assert per_w * NW == nodes and nchunk * chunk == per_w
    assert glen <= 128 and C % L == 0

    mesh = plsc.VectorSubcoreMesh(
        core_axis_name="c", subcore_axis_name="s",
        num_cores=NC, num_subcores=NS)

    @functools.partial(
        pl.kernel,
        out_type=jax.ShapeDtypeStruct((nodes, K, c2), jnp.float32),
        mesh=mesh,
        compiler_params=pltpu.CompilerParams(needs_layout_passes=False),
        scratch_types=[
            pltpu.VMEM((per_w * K,), jnp.int32),
            pltpu.VMEM((nchunk, glen), jnp.int32),
            pltpu.VMEM((glen, C), jnp.float32),
            pltpu.VMEM((glen, C), jnp.float32),
            pltpu.VMEM((chunk, K, c2), jnp.float32),
            pltpu.VMEM((chunk, K, c2), jnp.float32),
            pltpu.SemaphoreType.DMA,
            pltpu.SemaphoreType.DMA,
            pltpu.SemaphoreType.DMA,
            pltpu.SemaphoreType.DMA,
        ],
    )
    def ldg(points_hbm, idx_hbm, out_hbm,
            midx_v, cidx_v, g0_v, g1_v, o0_v, o1_v, gs0, gs1, ws0, ws1):
        g_v, o_v, gs, ws = (g0_v, g1_v), (o0_v, o1_v), (gs0, gs1), (ws0, ws1)
        wid = lax.axis_index("s") * NC + lax.axis_index("c")
        base0 = wid * per_w
        boff = jnp.where(base0 >= N, jnp.int32(N), jnp.int32(0))
        iot = lax.iota(jnp.int32, L)

        pltpu.sync_copy(idx_hbm.at[pl.ds(base0 * K, per_w * K)], midx_v)

        def build_body(ci, _):
            for r in range(chunk):
                cidx_v[ci, pl.ds(r * L, L)] = (
                    midx_v[pl.ds(ci * (chunk * K) + r * L, L)] + boff)
            cidx_v[ci, pl.ds(chunk * K, L)] = jnp.minimum(
                base0 + ci * chunk + iot, jnp.int32(nodes - 1))
            return 0

        lax.fori_loop(0, nchunk, build_body, 0)

        def fire_gather(ci, b):
            pltpu.async_copy(points_hbm.at[cidx_v.at[ci]], g_v[b], gs[b])

        def wait_gather(ci, b):
            pltpu.make_async_copy(
                points_hbm.at[cidx_v.at[ci]], g_v[b], gs[b]).wait()

        def out_slice(ci):
            return out_hbm.at[pl.ds(base0 + ci * chunk, chunk)]

        fire_gather(0, 0)

        def pair_body(it, _):
            for b in range(2):
                ci = it * 2 + b

                @pl.when(ci < nchunk)
                def _(ci=ci, b=b):
                    @pl.when(ci + 1 < nchunk)
                    def _():
                        fire_gather(ci + 1, 1 - b)

                    wait_gather(ci, b)

                    @pl.when(ci >= 2)
                    def _():
                        pltpu.make_async_copy(
                            o_v[b], out_slice(ci - 2), ws[b]).wait()

                    gb, ob = g_v[b], o_v[b]
                    for i in range(chunk):
                        xs = [gb[chunk * K + i, pl.ds(cc * L, L)]
                              for cc in range(C // L)]

                        @plsc.parallel_loop(0, K, unroll=4)
                        def _(j, i=i, xs=xs):
                            row = i * K + j
                            for cc in range(C // L):
                                gv = gb[row, pl.ds(cc * L, L)]
                                ob[i, j, pl.ds(cc * L, L)] = gv - xs[cc]
                                ob[i, j, pl.ds(C + cc * L, L)] = xs[cc]

                    pltpu.async_copy(o_v[b], out_slice(ci), ws[b])
            return 0

        lax.fori_loop(0, (nchunk + 1) // 2, pair_body, 0)
        pltpu.make_async_copy(o_v[1], out_slice(nchunk - 2), ws[1]).wait()
        pltpu.make_async_copy(o_v[0], out_slice(nchunk - 1), ws[0]).wait()

    return ldg


def kernel(points, idx):
    B, N, C = points.shape
    K = idx.shape[2]
    ldg = _build(B, N, C, K)
    out = ldg(points.reshape(B * N, C), idx.reshape(-1))
    return out.reshape(B, N, K, 2 * C).transpose(0, 1, 3, 2)


# speedup vs baseline: 13.3707x; 1.0261x over previous
import functools

import jax
import jax.numpy as jnp
from jax import lax
from jax.experimental import pallas as pl
from jax.experimental.pallas import tpu as pltpu
from jax.experimental.pallas import tpu_sc as plsc

NC, NS, L = 2, 16, 16
NW = NC * NS


@functools.lru_cache(maxsize=None)
def _build(B, N, C, K):
    nodes = B * N
    per_w = nodes // NW
    chunk = 5
    nchunk = per_w // chunk
    c2 = 2 * C
    glen = chunk * K + L
    assert per_w * NW == nodes and nchunk * chunk == per_w
    assert glen <= 128 and C % L == 0

    mesh = plsc.VectorSubcoreMesh(
        core_axis_name="c", subcore_axis_name="s",
        num_cores=NC, num_subcores=NS)

    @functools.partial(
        pl.kernel,
        out_type=jax.ShapeDtypeStruct((nodes, K, c2), jnp.float32),
        mesh=mesh,
        compiler_params=pltpu.CompilerParams(needs_layout_passes=False),
        scratch_types=[
            pltpu.VMEM((per_w * K,), jnp.int32),
            pltpu.VMEM((nchunk, glen), jnp.int32),
            pltpu.VMEM((glen, C), jnp.float32),
            pltpu.VMEM((glen, C), jnp.float32),
            pltpu.VMEM((chunk, K, c2), jnp.float32),
            pltpu.VMEM((chunk, K, c2), jnp.float32),
            pltpu.SemaphoreType.DMA,
            pltpu.SemaphoreType.DMA,
            pltpu.SemaphoreType.DMA,
            pltpu.SemaphoreType.DMA,
        ],
    )
    def ldg(points_hbm, idx_hbm, out_hbm,
            midx_v, cidx_v, g0_v, g1_v, o0_v, o1_v, gs0, gs1, ws0, ws1):
        g_v, o_v, gs, ws = (g0_v, g1_v), (o0_v, o1_v), (gs0, gs1), (ws0, ws1)
        wid = lax.axis_index("s") * NC + lax.axis_index("c")
        base0 = wid * per_w
        boff = jnp.where(base0 >= N, jnp.int32(N), jnp.int32(0))
        iot = lax.iota(jnp.int32, L)

        pltpu.sync_copy(idx_hbm.at[pl.ds(base0 * K, per_w * K)], midx_v)

        def build_body(ci, _):
            for r in range(chunk):
                cidx_v[ci, pl.ds(r * L, L)] = (
                    midx_v[pl.ds(ci * (chunk * K) + r * L, L)] + boff)
            cidx_v[ci, pl.ds(chunk * K, L)] = jnp.minimum(
                base0 + ci * chunk + iot, jnp.int32(nodes - 1))
            return 0

        lax.fori_loop(0, nchunk, build_body, 0)

        def fire_gather(ci, b):
            pltpu.async_copy(points_hbm.at[cidx_v.at[ci]], g_v[b], gs[b])

        def wait_gather(ci, b):
            pltpu.make_async_copy(
                points_hbm.at[cidx_v.at[ci]], g_v[b], gs[b]).wait()

        def out_slice(ci):
            return out_hbm.at[pl.ds(base0 + ci * chunk, chunk)]

        fire_gather(0, 0)

        def pair_body(it, _):
            for b in range(2):
                ci = it * 2 + b

                @pl.when(ci < nchunk)
                def _(ci=ci, b=b):
                    @pl.when(ci + 1 < nchunk)
                    def _():
                        fire_gather(ci + 1, 1 - b)

                    wait_gather(ci, b)

                    @pl.when(ci >= 2)
                    def _():
                        pltpu.make_async_copy(
                            o_v[b], out_slice(ci - 2), ws[b]).wait()

                    gb, ob = g_v[b], o_v[b]
                    for i in range(chunk):
                        xs = [gb[chunk * K + i, pl.ds(cc * L, L)]
                              for cc in range(C // L)]

                        @plsc.parallel_loop(0, K, unroll=8)
                        def _(j, i=i, xs=xs):
                            row = i * K + j
                            for cc in range(C // L):
                                gv = gb[row, pl.ds(cc * L, L)]
                                ob[i, j, pl.ds(cc * L, L)] = gv - xs[cc]
                                ob[i, j, pl.ds(C + cc * L, L)] = xs[cc]

                    pltpu.async_copy(o_v[b], out_slice(ci), ws[b])
            return 0

        lax.fori_loop(0, (nchunk + 1) // 2, pair_body, 0)
        pltpu.make_async_copy(o_v[1], out_slice(nchunk - 2), ws[1]).wait()
        pltpu.make_async_copy(o_v[0], out_slice(nchunk - 1), ws[0]).wait()

    return ldg


def kernel(points, idx):
    B, N, C = points.shape
    K = idx.shape[2]
    ldg = _build(B, N, C, K)
    out = ldg(points.reshape(B * N, C), idx.reshape(-1))
    return out.reshape(B, N, K, 2 * C).transpose(0, 1, 3, 2)


# speedup vs baseline: 13.7172x; 1.0259x over previous
import functools

import jax
import jax.numpy as jnp
from jax import lax
from jax.experimental import pallas as pl
from jax.experimental.pallas import tpu as pltpu
from jax.experimental.pallas import tpu_sc as plsc

NC, NS, L = 2, 16, 16
NW = NC * NS


@functools.lru_cache(maxsize=None)
def _build(B, N, C, K):
    nodes = B * N
    per_w = nodes // NW
    chunk = 5
    nchunk = per_w // chunk
    c2 = 2 * C
    glen = chunk * K + 8
    assert per_w * NW == nodes and nchunk * chunk == per_w
    assert glen <= 128 and glen % 8 == 0 and C % L == 0 and chunk + 3 <= L

    mesh = plsc.VectorSubcoreMesh(
        core_axis_name="c", subcore_axis_name="s",
        num_cores=NC, num_subcores=NS)

    @functools.partial(
        pl.kernel,
        out_type=jax.ShapeDtypeStruct((nodes, K, c2), jnp.float32),
        mesh=mesh,
        compiler_params=pltpu.CompilerParams(needs_layout_passes=False),
        scratch_types=[
            pltpu.VMEM((per_w * K,), jnp.int32),
            pltpu.VMEM((nchunk, glen), jnp.int32),
            pltpu.VMEM((glen, C), jnp.float32),
            pltpu.VMEM((glen, C), jnp.float32),
            pltpu.VMEM((chunk, K, c2), jnp.float32),
            pltpu.VMEM((chunk, K, c2), jnp.float32),
            pltpu.SemaphoreType.DMA,
            pltpu.SemaphoreType.DMA,
            pltpu.SemaphoreType.DMA,
            pltpu.SemaphoreType.DMA,
        ],
    )
    def ldg(points_hbm, idx_hbm, out_hbm,
            midx_v, cidx_v, g0_v, g1_v, o0_v, o1_v, gs0, gs1, ws0, ws1):
        g_v, o_v, gs, ws = (g0_v, g1_v), (o0_v, o1_v), (gs0, gs1), (ws0, ws1)
        wid = lax.axis_index("s") * NC + lax.axis_index("c")
        base0 = wid * per_w
        boff = jnp.where(base0 >= N, jnp.int32(N), jnp.int32(0))
        iot = lax.iota(jnp.int32, L)

        pltpu.sync_copy(idx_hbm.at[pl.ds(base0 * K, per_w * K)], midx_v)

        def build_body(ci, _):
            for r in range(chunk):
                cidx_v[ci, pl.ds(r * L, L)] = (
                    midx_v[pl.ds(ci * (chunk * K) + r * L, L)] + boff)
            plsc.store_scatter(
                cidx_v,
                [jnp.full((L,), ci, jnp.int32),
                 jnp.minimum(chunk * K + iot, jnp.int32(glen - 1))],
                jnp.minimum(base0 + ci * chunk + iot, jnp.int32(nodes - 1)))
            return 0

        lax.fori_loop(0, nchunk, build_body, 0)

        def fire_gather(ci, b):
            pltpu.async_copy(points_hbm.at[cidx_v.at[ci]], g_v[b], gs[b])

        def wait_gather(ci, b):
            pltpu.make_async_copy(
                points_hbm.at[cidx_v.at[ci]], g_v[b], gs[b]).wait()

        def out_slice(ci):
            return out_hbm.at[pl.ds(base0 + ci * chunk, chunk)]

        fire_gather(0, 0)

        def pair_body(it, _):
            for b in range(2):
                ci = it * 2 + b

                @pl.when(ci < nchunk)
                def _(ci=ci, b=b):
                    @pl.when(ci + 1 < nchunk)
                    def _():
                        fire_gather(ci + 1, 1 - b)

                    wait_gather(ci, b)

                    @pl.when(ci >= 2)
                    def _():
                        pltpu.make_async_copy(
                            o_v[b], out_slice(ci - 2), ws[b]).wait()

                    gb, ob = g_v[b], o_v[b]
                    for i in range(chunk):
                        xs = [gb[chunk * K + i, pl.ds(cc * L, L)]
                              for cc in range(C // L)]

                        @plsc.parallel_loop(0, K, unroll=8)
                        def _(j, i=i, xs=xs):
                            row = i * K + j
                            for cc in range(C // L):
                                gv = gb[row, pl.ds(cc * L, L)]
                                ob[i, j, pl.ds(cc * L, L)] = gv - xs[cc]
                                ob[i, j, pl.ds(C + cc * L, L)] = xs[cc]

                    pltpu.async_copy(o_v[b], out_slice(ci), ws[b])
            return 0

        lax.fori_loop(0, (nchunk + 1) // 2, pair_body, 0)
        pltpu.make_async_copy(o_v[1], out_slice(nchunk - 2), ws[1]).wait()
        pltpu.make_async_copy(o_v[0], out_slice(nchunk - 1), ws[0]).wait()

    return ldg


def kernel(points, idx):
    B, N, C = points.shape
    K = idx.shape[2]
    ldg = _build(B, N, C, K)
    out = ldg(points.reshape(B * N, C), idx.reshape(-1))
    return out.reshape(B, N, K, 2 * C).transpose(0, 1, 3, 2)
